# baseline (speedup 1.0000x reference)
import functools

import jax
import jax.numpy as jnp
from jax import lax
from jax.experimental import pallas as pl
from jax.experimental.pallas import tpu as pltpu
from jax.experimental.pallas import tpu_sc as plsc

_NUM_CLASS = 80
_STRIDES = (32.0, 16.0, 8.0)
_ANCHORS = (
    ((116.0, 90.0), (156.0, 198.0), (373.0, 326.0)),
    ((30.0, 61.0), (62.0, 45.0), (59.0, 119.0)),
    ((10.0, 13.0), (16.0, 30.0), (33.0, 23.0)),
)
_GRIDS = (19, 38, 76)
_B = 16
_NT = 128
_NO = 85
_NR = 3 * _NT
_NRP = 512
_NC, _NS = 2, 16
_NW = _NC * _NS
_RPW = _NRP // _NW


def _scale_targets(tgt, s):
    g = float(_GRIDS[s])
    stride = _STRIDES[s]
    b = tgt[:, 0:1]
    c = tgt[:, 1:2]
    gx = tgt[:, 2:3] * g
    gy = tgt[:, 3:4] * g
    tw = tgt[:, 4:5] * g
    th = tgt[:, 5:6] * g
    fx = jnp.floor(gx)
    fy = jnp.floor(gy)
    gi = jnp.clip(fx, 0.0, g - 1.0)
    gj = jnp.clip(fy, 0.0, g - 1.0)
    tbx = gx - fx
    tby = gy - fy
    ones = jnp.ones_like(b)
    per_anchor = []
    for a in range(3):
        aw = _ANCHORS[s][a][0] / stride
        ah = _ANCHORS[s][a][1] / stride
        rw = tw / aw
        rh = th / ah
        rmax = jnp.maximum(jnp.maximum(rw, 1.0 / rw), jnp.maximum(rh, 1.0 / rh))
        m = (rmax < 4.0).astype(jnp.float32)
        fidx = ((b * 3.0 + float(a)) * g + gj) * g + gi
        per_anchor.append((m, fidx, tbx, tby, tw, th, aw * ones, ah * ones, c,
                           b, a * ones, gj, gi))
    return [jnp.concatenate([pa[k] for pa in per_anchor], axis=0)
            for k in range(13)]


def _prep_body(tgt_ref, *orefs):
    tgt = tgt_ref[:, :]
    pad = jnp.zeros((_NRP - _NR, 1), jnp.float32)
    for s in range(3):
        g = float(_GRIDS[s])
        vals = _scale_targets(tgt, s)
        b, a, gj, gi = vals[9], vals[10], vals[11], vals[12]
        srow = (b * 3.0 + a) * g + gj
        for k, v in enumerate((srow, gi)):
            orefs[2 * s + k][:, :] = jnp.concatenate(
                [v, pad], axis=0).astype(jnp.int32)


def _prep(target):
    return pl.pallas_call(
        _prep_body,
        out_shape=[jax.ShapeDtypeStruct((_NRP, 1), jnp.int32)] * 6,
    )(target)


_NSL = (48 * 19, 48 * 38, 48 * 76)
_CH4_WORKERS = (16, 32, 32)
_CH4_PER_W = (_NSL[0] // 16, _NSL[1] // 32, _NSL[2] // 32)


def _sc_gather_body(t0, t1, t2, *rest):
    idx_refs = rest[:6]
    outs = rest[6:9]
    sv, gv, rows_v, sem = rest[9:13]
    wid = lax.axis_index("s") * _NC + lax.axis_index("c")
    tabs = (t0, t1, t2)

    base = wid * _RPW
    for s in range(3):
        isr, igi = idx_refs[2 * s: 2 * s + 2]
        pltpu.sync_copy(isr.at[pl.ds(base, _RPW)], sv)
        pltpu.sync_copy(igi.at[pl.ds(base, _RPW)], gv)
        ss = sv[...]
        gg = gv[...]
        copies = []
        for r in range(_RPW):
            copies.append(pltpu.async_copy(
                tabs[s].at[ss[r], gg[r], :], rows_v.at[r, :], sem))
        for cp in copies:
            cp.wait()
        pltpu.sync_copy(rows_v, outs[s].at[pl.ds(base, _RPW), :])


@functools.cache
def _sc_gather_kernel():
    return pl.kernel(
        _sc_gather_body,
        out_type=[jax.ShapeDtypeStruct((_NRP, _NO), jnp.float32)] * 3,
        mesh=plsc.VectorSubcoreMesh(core_axis_name="c", subcore_axis_name="s",
                                    num_cores=_NC, num_subcores=_NS),
        scratch_types=[
            pltpu.VMEM((_RPW,), jnp.int32),
            pltpu.VMEM((_RPW,), jnp.int32),
            pltpu.VMEM((_RPW, _NO), jnp.float32),
            pltpu.SemaphoreType.DMA,
        ],
    )


def _sc_gather(p0, p1, p2, idxs):
    return _sc_gather_kernel()(p0, p1, p2, *idxs)


_GJC = (1, 2, 4)


def _obj_body(p0_ref, p1a_ref, p1b_ref, p1c_ref, p2a_ref, p2b_ref,
              p2c_ref, s0_ref, s1_ref, s2_ref):
    step = pl.program_id(0)

    def _sp_sum(x4):
        return jnp.sum(jnp.maximum(x4, 0.0) + jnp.log1p(jnp.exp(-jnp.abs(x4))))

    x = p0_ref[0, :, :, :, :]
    part0 = _sp_sum(jnp.swapaxes(x, -1, -2)[:, :, 4, :])
    part1 = 0.0
    for r in (p1a_ref, p1b_ref, p1c_ref):
        x = r[0, 0, :, :, :]
        part1 = part1 + _sp_sum(jnp.swapaxes(x, -1, -2)[:, 4, :])
    part2 = 0.0
    for r in (p2a_ref, p2b_ref, p2c_ref):
        x = r[0, 0, :, :, :]
        part2 = part2 + _sp_sum(jnp.swapaxes(x, -1, -2)[:, 4, :])
    for s_ref, part in ((s0_ref, part0), (s1_ref, part1), (s2_ref, part2)):
        p11 = jnp.broadcast_to(part, (1, 1))

        @pl.when(step == 0)
        def _init(s_ref=s_ref, p11=p11):
            s_ref[:, :] = p11

        @pl.when(step != 0)
        def _acc(s_ref=s_ref, p11=p11):
            s_ref[:, :] += p11


def _obj_sums(pred0, pred1, pred2):
    specs = [pl.BlockSpec((1, 3, 19, 19, _NO), lambda t: (t, 0, 0, 0, 0))]
    for g in (38, 76):
        for a in range(3):
            specs.append(pl.BlockSpec(
                (1, 1, g, g, _NO),
                lambda t, a=a: (t, a, 0, 0, 0)))
    return pl.pallas_call(
        _obj_body,
        grid=(_B,),
        in_specs=specs,
        out_specs=[pl.BlockSpec((1, 1), lambda t: (0, 0))] * 3,
        out_shape=[jax.ShapeDtypeStruct((1, 1), jnp.float32)] * 3,
    )(pred0, pred1, pred1, pred1, pred2, pred2, pred2)


def _final_body(tgt_ref, c0_ref, c1_ref, c2_ref, ps0_ref, ps1_ref, ps2_ref,
                o_ref):
    tgt = tgt_ref[:, :]
    lcls = 0.0
    lobj = 0.0
    lbox = 0.0
    for s, c_ref, ps_ref in ((0, c0_ref, ps0_ref), (1, c1_ref, ps1_ref),
                             (2, c2_ref, ps2_ref)):
        ncell = float(_B * 3 * _GRIDS[s] * _GRIDS[s])
        m, fidx, tbx, tby, tw, th, aw, ah, c = _scale_targets(tgt, s)[:9]
        ps = ps_ref[0:_NR, :]
        denom = jnp.maximum(jnp.sum(m), 1.0)
        px = jax.nn.sigmoid(ps[:, 0:1])
        py = jax.nn.sigmoid(ps[:, 1:2])
        pw = jnp.exp(ps[:, 2:3]) * aw
        ph = jnp.exp(ps[:, 3:4]) * ah
        b1x1 = px - pw * 0.5
        b1x2 = px + pw * 0.5
        b1y1 = py - ph * 0.5
        b1y2 = py + ph * 0.5
        b2x1 = tbx - tw * 0.5
        b2x2 = tbx + tw * 0.5
        b2y1 = tby - th * 0.5
        b2y2 = tby + th * 0.5
        iw = jnp.maximum(jnp.minimum(b1x2, b2x2) - jnp.maximum(b1x1, b2x1), 0.0)
        ih = jnp.maximum(jnp.minimum(b1y2, b2y2) - jnp.maximum(b1y1, b2y1), 0.0)
        inter = iw * ih
        w1 = b1x2 - b1x1
        h1 = b1y2 - b1y1
        w2 = b2x2 - b2x1
        h2 = b2y2 - b2y1
        union = w1 * h1 + w2 * h2 - inter + 1e-9
        iou = inter / union
        lbox = lbox + jnp.sum(m * (1.0 - iou)) / denom
        xc = ps[:, 5:_NO]
        cls_iota = lax.broadcasted_iota(
            jnp.int32, (_NR, _NUM_CLASS), 1).astype(jnp.float32)
        oh_cls = (cls_iota == c).astype(jnp.float32)
        e = jnp.maximum(xc, 0.0) - xc * oh_cls + jnp.log1p(jnp.exp(-jnp.abs(xc)))
        lcls = lcls + jnp.sum(m * e) / (denom * _NUM_CLASS)
        row_iota = lax.broadcasted_iota(jnp.int32, (_NR, 1), 0).astype(
            jnp.float32)
        key = jnp.where(m > 0.0, fidx, -(row_iota + 1.0))
        ones = jnp.ones((_NR, 1), jnp.float32)
        keyrow = lax.dot_general(ones, key, (((1,), (1,)), ((), ())),
                                 precision=lax.Precision.HIGHEST,
                                 preferred_element_type=jnp.float32)
        ii = lax.broadcasted_iota(jnp.int32, (_NR, _NR), 0)
        jj = lax.broadcasted_iota(jnp.int32, (_NR, _NR), 1)
        later_dup = jnp.where((keyrow == key) & (jj > ii), 1.0, 0.0)
        loser = jnp.max(later_dup, axis=1, keepdims=True)
        v = jnp.maximum(iou, 0.0)
        corr = jnp.sum(m * (1.0 - loser) * ps[:, 4:5] * v)
        lobj = lobj + (c_ref[0, 0] - corr) / ncell
    loss = 0.05 * lcls + lobj + 0.5 * lbox
    o_ref[:, :] = jnp.broadcast_to(loss, (1, 1))


def _final(target, c0, c1, c2, ps0, ps1, ps2):
    return pl.pallas_call(
        _final_body,
        out_shape=jax.ShapeDtypeStruct((1, 1), jnp.float32),
    )(target, c0, c1, c2, ps0, ps1, ps2)


def kernel(pred0, pred1, pred2, target):
    views = [p.reshape(_NSL[s], _GRIDS[s], _NO)
             for s, p in enumerate((pred0, pred1, pred2))]
    idxs = [i.reshape(_NRP) for i in _prep(target)]
    ps0, ps1, ps2 = _sc_gather(*views, idxs)
    s0, s1, s2 = _obj_sums(pred0, pred1, pred2)
    out = _final(target, s0, s1, s2, ps0, ps1, ps2)
    return out.reshape(1)


# speedup vs baseline: 1.0007x; 1.0007x over previous
import functools

import jax
import jax.numpy as jnp
from jax import lax
from jax.experimental import pallas as pl
from jax.experimental.pallas import tpu as pltpu
from jax.experimental.pallas import tpu_sc as plsc

_NUM_CLASS = 80
_STRIDES = (32.0, 16.0, 8.0)
_ANCHORS = (
    ((116.0, 90.0), (156.0, 198.0), (373.0, 326.0)),
    ((30.0, 61.0), (62.0, 45.0), (59.0, 119.0)),
    ((10.0, 13.0), (16.0, 30.0), (33.0, 23.0)),
)
_GRIDS = (19, 38, 76)
_B = 16
_NT = 128
_NO = 85
_NR = 3 * _NT
_NRP = 512
_NC, _NS = 2, 16
_NW = _NC * _NS
_RPW = _NRP // _NW


def _scale_targets(tgt, s):
    g = float(_GRIDS[s])
    stride = _STRIDES[s]
    b = tgt[:, 0:1]
    c = tgt[:, 1:2]
    gx = tgt[:, 2:3] * g
    gy = tgt[:, 3:4] * g
    tw = tgt[:, 4:5] * g
    th = tgt[:, 5:6] * g
    fx = jnp.floor(gx)
    fy = jnp.floor(gy)
    gi = jnp.clip(fx, 0.0, g - 1.0)
    gj = jnp.clip(fy, 0.0, g - 1.0)
    tbx = gx - fx
    tby = gy - fy
    ones = jnp.ones_like(b)
    per_anchor = []
    for a in range(3):
        aw = _ANCHORS[s][a][0] / stride
        ah = _ANCHORS[s][a][1] / stride
        rw = tw / aw
        rh = th / ah
        rmax = jnp.maximum(jnp.maximum(rw, 1.0 / rw), jnp.maximum(rh, 1.0 / rh))
        m = (rmax < 4.0).astype(jnp.float32)
        fidx = ((b * 3.0 + float(a)) * g + gj) * g + gi
        per_anchor.append((m, fidx, tbx, tby, tw, th, aw * ones, ah * ones, c,
                           b, a * ones, gj, gi))
    return [jnp.concatenate([pa[k] for pa in per_anchor], axis=0)
            for k in range(13)]


def _prep_body(tgt_ref, *orefs):
    tgt = tgt_ref[:, :]
    pad = jnp.zeros((_NRP - _NR, 1), jnp.float32)
    for s in range(3):
        g = float(_GRIDS[s])
        vals = _scale_targets(tgt, s)
        b, a, gj, gi = vals[9], vals[10], vals[11], vals[12]
        srow = (b * 3.0 + a) * g + gj
        for k, v in enumerate((srow, gi)):
            orefs[2 * s + k][:, :] = jnp.concatenate(
                [v, pad], axis=0).astype(jnp.int32)


def _prep(target):
    return pl.pallas_call(
        _prep_body,
        out_shape=[jax.ShapeDtypeStruct((_NRP, 1), jnp.int32)] * 6,
    )(target)


_NSL = (48 * 19, 48 * 38, 48 * 76)
_CH4_WORKERS = (16, 32, 32)
_CH4_PER_W = (_NSL[0] // 16, _NSL[1] // 32, _NSL[2] // 32)


def _sc_gather_body(t0, t1, t2, *rest):
    idx_refs = rest[:6]
    outs = rest[6:9]
    svs = rest[9:12]
    gvs = rest[12:15]
    rows = rest[15:18]
    semi, sem, semo = rest[18:21]
    wid = lax.axis_index("s") * _NC + lax.axis_index("c")
    tabs = (t0, t1, t2)
    base = wid * _RPW
    ijobs = []
    for s in range(3):
        ijobs.append(pltpu.async_copy(
            idx_refs[2 * s].at[pl.ds(base, _RPW)], svs[s], semi))
        ijobs.append(pltpu.async_copy(
            idx_refs[2 * s + 1].at[pl.ds(base, _RPW)], gvs[s], semi))
    for j in ijobs:
        j.wait()
    rjobs = []
    for s in range(3):
        ss = svs[s][...]
        gg = gvs[s][...]
        for r in range(_RPW):
            rjobs.append(pltpu.async_copy(
                tabs[s].at[ss[r], gg[r], :], rows[s].at[r, :], sem))
    for j in rjobs:
        j.wait()
    ojobs = []
    for s in range(3):
        ojobs.append(pltpu.async_copy(
            rows[s], outs[s].at[pl.ds(base, _RPW), :], semo))
    for j in ojobs:
        j.wait()


@functools.cache
def _sc_gather_kernel():
    return pl.kernel(
        _sc_gather_body,
        out_type=[jax.ShapeDtypeStruct((_NRP, _NO), jnp.float32)] * 3,
        mesh=plsc.VectorSubcoreMesh(core_axis_name="c", subcore_axis_name="s",
                                    num_cores=_NC, num_subcores=_NS),
        scratch_types=(
            [pltpu.VMEM((_RPW,), jnp.int32)] * 6
            + [pltpu.VMEM((_RPW, _NO), jnp.float32)] * 3
            + [pltpu.SemaphoreType.DMA] * 3
        ),
    )


def _sc_gather(p0, p1, p2, idxs):
    return _sc_gather_kernel()(p0, p1, p2, *idxs)


_GJC = (1, 2, 4)


def _obj_body(p0_ref, p1a_ref, p1b_ref, p1c_ref, p2a_ref, p2b_ref,
              p2c_ref, s0_ref, s1_ref, s2_ref):
    step = pl.program_id(0)

    def _sp_sum(x4):
        return jnp.sum(jnp.maximum(x4, 0.0) + jnp.log1p(jnp.exp(-jnp.abs(x4))))

    x = p0_ref[0, :, :, :, :]
    part0 = _sp_sum(jnp.swapaxes(x, -1, -2)[:, :, 4, :])
    part1 = 0.0
    for r in (p1a_ref, p1b_ref, p1c_ref):
        x = r[0, 0, :, :, :]
        part1 = part1 + _sp_sum(jnp.swapaxes(x, -1, -2)[:, 4, :])
    part2 = 0.0
    for r in (p2a_ref, p2b_ref, p2c_ref):
        x = r[0, 0, :, :, :]
        part2 = part2 + _sp_sum(jnp.swapaxes(x, -1, -2)[:, 4, :])
    for s_ref, part in ((s0_ref, part0), (s1_ref, part1), (s2_ref, part2)):
        p11 = jnp.broadcast_to(part, (1, 1))

        @pl.when(step == 0)
        def _init(s_ref=s_ref, p11=p11):
            s_ref[:, :] = p11

        @pl.when(step != 0)
        def _acc(s_ref=s_ref, p11=p11):
            s_ref[:, :] += p11


def _obj_sums(pred0, pred1, pred2):
    specs = [pl.BlockSpec((1, 3, 19, 19, _NO), lambda t: (t, 0, 0, 0, 0))]
    for g in (38, 76):
        for a in range(3):
            specs.append(pl.BlockSpec(
                (1, 1, g, g, _NO),
                lambda t, a=a: (t, a, 0, 0, 0)))
    return pl.pallas_call(
        _obj_body,
        grid=(_B,),
        in_specs=specs,
        out_specs=[pl.BlockSpec((1, 1), lambda t: (0, 0))] * 3,
        out_shape=[jax.ShapeDtypeStruct((1, 1), jnp.float32)] * 3,
    )(pred0, pred1, pred1, pred1, pred2, pred2, pred2)


def _final_body(tgt_ref, c0_ref, c1_ref, c2_ref, ps0_ref, ps1_ref, ps2_ref,
                o_ref):
    tgt = tgt_ref[:, :]
    lcls = 0.0
    lobj = 0.0
    lbox = 0.0
    for s, c_ref, ps_ref in ((0, c0_ref, ps0_ref), (1, c1_ref, ps1_ref),
                             (2, c2_ref, ps2_ref)):
        ncell = float(_B * 3 * _GRIDS[s] * _GRIDS[s])
        m, fidx, tbx, tby, tw, th, aw, ah, c = _scale_targets(tgt, s)[:9]
        ps = ps_ref[0:_NR, :]
        denom = jnp.maximum(jnp.sum(m), 1.0)
        px = jax.nn.sigmoid(ps[:, 0:1])
        py = jax.nn.sigmoid(ps[:, 1:2])
        pw = jnp.exp(ps[:, 2:3]) * aw
        ph = jnp.exp(ps[:, 3:4]) * ah
        b1x1 = px - pw * 0.5
        b1x2 = px + pw * 0.5
        b1y1 = py - ph * 0.5
        b1y2 = py + ph * 0.5
        b2x1 = tbx - tw * 0.5
        b2x2 = tbx + tw * 0.5
        b2y1 = tby - th * 0.5
        b2y2 = tby + th * 0.5
        iw = jnp.maximum(jnp.minimum(b1x2, b2x2) - jnp.maximum(b1x1, b2x1), 0.0)
        ih = jnp.maximum(jnp.minimum(b1y2, b2y2) - jnp.maximum(b1y1, b2y1), 0.0)
        inter = iw * ih
        w1 = b1x2 - b1x1
        h1 = b1y2 - b1y1
        w2 = b2x2 - b2x1
        h2 = b2y2 - b2y1
        union = w1 * h1 + w2 * h2 - inter + 1e-9
        iou = inter / union
        lbox = lbox + jnp.sum(m * (1.0 - iou)) / denom
        xc = ps[:, 5:_NO]
        cls_iota = lax.broadcasted_iota(
            jnp.int32, (_NR, _NUM_CLASS), 1).astype(jnp.float32)
        oh_cls = (cls_iota == c).astype(jnp.float32)
        e = jnp.maximum(xc, 0.0) - xc * oh_cls + jnp.log1p(jnp.exp(-jnp.abs(xc)))
        lcls = lcls + jnp.sum(m * e) / (denom * _NUM_CLASS)
        row_iota = lax.broadcasted_iota(jnp.int32, (_NR, 1), 0).astype(
            jnp.float32)
        key = jnp.where(m > 0.0, fidx, -(row_iota + 1.0))
        ones = jnp.ones((_NR, 1), jnp.float32)
        keyrow = lax.dot_general(ones, key, (((1,), (1,)), ((), ())),
                                 precision=lax.Precision.HIGHEST,
                                 preferred_element_type=jnp.float32)
        ii = lax.broadcasted_iota(jnp.int32, (_NR, _NR), 0)
        jj = lax.broadcasted_iota(jnp.int32, (_NR, _NR), 1)
        later_dup = jnp.where((keyrow == key) & (jj > ii), 1.0, 0.0)
        loser = jnp.max(later_dup, axis=1, keepdims=True)
        v = jnp.maximum(iou, 0.0)
        corr = jnp.sum(m * (1.0 - loser) * ps[:, 4:5] * v)
        lobj = lobj + (c_ref[0, 0] - corr) / ncell
    loss = 0.05 * lcls + lobj + 0.5 * lbox
    o_ref[:, :] = jnp.broadcast_to(loss, (1, 1))


def _final(target, c0, c1, c2, ps0, ps1, ps2):
    return pl.pallas_call(
        _final_body,
        out_shape=jax.ShapeDtypeStruct((1, 1), jnp.float32),
    )(target, c0, c1, c2, ps0, ps1, ps2)


def kernel(pred0, pred1, pred2, target):
    views = [p.reshape(_NSL[s], _GRIDS[s], _NO)
             for s, p in enumerate((pred0, pred1, pred2))]
    idxs = [i.reshape(_NRP) for i in _prep(target)]
    ps0, ps1, ps2 = _sc_gather(*views, idxs)
    s0, s1, s2 = _obj_sums(pred0, pred1, pred2)
    out = _final(target, s0, s1, s2, ps0, ps1, ps2)
    return out.reshape(1)


# speedup vs baseline: 1.0258x; 1.0251x over previous
import functools

import jax
import jax.numpy as jnp
from jax import lax
from jax.experimental import pallas as pl
from jax.experimental.pallas import tpu as pltpu
from jax.experimental.pallas import tpu_sc as plsc

_NUM_CLASS = 80
_STRIDES = (32.0, 16.0, 8.0)
_ANCHORS = (
    ((116.0, 90.0), (156.0, 198.0), (373.0, 326.0)),
    ((30.0, 61.0), (62.0, 45.0), (59.0, 119.0)),
    ((10.0, 13.0), (16.0, 30.0), (33.0, 23.0)),
)
_GRIDS = (19, 38, 76)
_B = 16
_NT = 128
_NO = 85
_NR = 3 * _NT
_NRP = 512
_NC, _NS = 2, 16
_NW = _NC * _NS
_RPW = _NRP // _NW


def _scale_targets(tgt, s):
    g = float(_GRIDS[s])
    stride = _STRIDES[s]
    b = tgt[:, 0:1]
    c = tgt[:, 1:2]
    gx = tgt[:, 2:3] * g
    gy = tgt[:, 3:4] * g
    tw = tgt[:, 4:5] * g
    th = tgt[:, 5:6] * g
    fx = jnp.floor(gx)
    fy = jnp.floor(gy)
    gi = jnp.clip(fx, 0.0, g - 1.0)
    gj = jnp.clip(fy, 0.0, g - 1.0)
    tbx = gx - fx
    tby = gy - fy
    ones = jnp.ones_like(b)
    per_anchor = []
    for a in range(3):
        aw = _ANCHORS[s][a][0] / stride
        ah = _ANCHORS[s][a][1] / stride
        rw = tw / aw
        rh = th / ah
        rmax = jnp.maximum(jnp.maximum(rw, 1.0 / rw), jnp.maximum(rh, 1.0 / rh))
        m = (rmax < 4.0).astype(jnp.float32)
        fidx = ((b * 3.0 + float(a)) * g + gj) * g + gi
        per_anchor.append((m, fidx, tbx, tby, tw, th, aw * ones, ah * ones, c,
                           b, a * ones, gj, gi))
    return [jnp.concatenate([pa[k] for pa in per_anchor], axis=0)
            for k in range(13)]


def _prep_body(tgt_ref, *orefs):
    tgt = tgt_ref[:, :]
    pad = jnp.zeros((_NRP - _NR, 1), jnp.float32)
    for s in range(3):
        g = float(_GRIDS[s])
        vals = _scale_targets(tgt, s)
        b, a, gj, gi = vals[9], vals[10], vals[11], vals[12]
        srow = (b * 3.0 + a) * g + gj
        for k, v in enumerate((srow, gi)):
            orefs[2 * s + k][:, :] = jnp.concatenate(
                [v, pad], axis=0).astype(jnp.int32)


def _prep(target):
    return pl.pallas_call(
        _prep_body,
        out_shape=[jax.ShapeDtypeStruct((_NRP, 1), jnp.int32)] * 6,
    )(target)


_NSL = (48 * 19, 48 * 38, 48 * 76)
_CH4_WORKERS = (16, 32, 32)
_CH4_PER_W = (_NSL[0] // 16, _NSL[1] // 32, _NSL[2] // 32)


def _sc_gather_body(t0, t1, t2, *rest):
    idx_refs = rest[:6]
    outs = rest[6:9]
    svs = rest[9:12]
    gvs = rest[12:15]
    rows = rest[15:18]
    semi, sem, semo = rest[18:21]
    wid = lax.axis_index("s") * _NC + lax.axis_index("c")
    tabs = (t0, t1, t2)
    base = wid * _RPW

    @pl.when(wid < _NR // _RPW)
    def _work():
        _sc_gather_work(tabs, idx_refs, outs, svs, gvs, rows, semi, sem, semo,
                        base)


def _sc_gather_work(tabs, idx_refs, outs, svs, gvs, rows, semi, sem, semo,
                    base):
    ijobs = []
    for s in range(3):
        ijobs.append(pltpu.async_copy(
            idx_refs[2 * s].at[pl.ds(base, _RPW)], svs[s], semi))
        ijobs.append(pltpu.async_copy(
            idx_refs[2 * s + 1].at[pl.ds(base, _RPW)], gvs[s], semi))
    for j in ijobs:
        j.wait()
    rjobs = []
    for s in range(3):
        ss = svs[s][...]
        gg = gvs[s][...]
        for r in range(_RPW):
            rjobs.append(pltpu.async_copy(
                tabs[s].at[ss[r], gg[r], :], rows[s].at[r, :], sem))
    for j in rjobs:
        j.wait()
    ojobs = []
    for s in range(3):
        ojobs.append(pltpu.async_copy(
            rows[s], outs[s].at[pl.ds(base, _RPW), :], semo))
    for j in ojobs:
        j.wait()


@functools.cache
def _sc_gather_kernel():
    return pl.kernel(
        _sc_gather_body,
        out_type=[jax.ShapeDtypeStruct((_NRP, _NO), jnp.float32)] * 3,
        mesh=plsc.VectorSubcoreMesh(core_axis_name="c", subcore_axis_name="s",
                                    num_cores=_NC, num_subcores=_NS),
        scratch_types=(
            [pltpu.VMEM((_RPW,), jnp.int32)] * 6
            + [pltpu.VMEM((_RPW, _NO), jnp.float32)] * 3
            + [pltpu.SemaphoreType.DMA] * 3
        ),
    )


def _sc_gather(p0, p1, p2, idxs):
    return _sc_gather_kernel()(p0, p1, p2, *idxs)


_GJC = (1, 2, 4)


def _obj_body(p0_ref, p1a_ref, p1b_ref, p1c_ref, p2a_ref, p2b_ref,
              p2c_ref, s0_ref, s1_ref, s2_ref):
    step = pl.program_id(0)

    def _sp_sum(x4):
        return jnp.sum(jnp.maximum(x4, 0.0) + jnp.log1p(jnp.exp(-jnp.abs(x4))))

    x = p0_ref[0, :, :, :, :]
    part0 = _sp_sum(jnp.swapaxes(x, -1, -2)[:, :, 4, :])
    part1 = 0.0
    for r in (p1a_ref, p1b_ref, p1c_ref):
        x = r[0, 0, :, :, :]
        part1 = part1 + _sp_sum(jnp.swapaxes(x, -1, -2)[:, 4, :])
    part2 = 0.0
    for r in (p2a_ref, p2b_ref, p2c_ref):
        x = r[0, 0, :, :, :]
        part2 = part2 + _sp_sum(jnp.swapaxes(x, -1, -2)[:, 4, :])
    for s_ref, part in ((s0_ref, part0), (s1_ref, part1), (s2_ref, part2)):
        p11 = jnp.broadcast_to(part, (1, 1))

        @pl.when(step == 0)
        def _init(s_ref=s_ref, p11=p11):
            s_ref[:, :] = p11

        @pl.when(step != 0)
        def _acc(s_ref=s_ref, p11=p11):
            s_ref[:, :] += p11


def _obj_sums(pred0, pred1, pred2):
    specs = [pl.BlockSpec((1, 3, 19, 19, _NO), lambda t: (t, 0, 0, 0, 0))]
    for g in (38, 76):
        for a in range(3):
            specs.append(pl.BlockSpec(
                (1, 1, g, g, _NO),
                lambda t, a=a: (t, a, 0, 0, 0)))
    return pl.pallas_call(
        _obj_body,
        grid=(_B,),
        in_specs=specs,
        out_specs=[pl.BlockSpec((1, 1), lambda t: (0, 0))] * 3,
        out_shape=[jax.ShapeDtypeStruct((1, 1), jnp.float32)] * 3,
    )(pred0, pred1, pred1, pred1, pred2, pred2, pred2)


def _final_body(tgt_ref, c0_ref, c1_ref, c2_ref, ps0_ref, ps1_ref, ps2_ref,
                o_ref):
    tgt = tgt_ref[:, :]
    lcls = 0.0
    lobj = 0.0
    lbox = 0.0
    for s, c_ref, ps_ref in ((0, c0_ref, ps0_ref), (1, c1_ref, ps1_ref),
                             (2, c2_ref, ps2_ref)):
        ncell = float(_B * 3 * _GRIDS[s] * _GRIDS[s])
        m, fidx, tbx, tby, tw, th, aw, ah, c = _scale_targets(tgt, s)[:9]
        ps = ps_ref[0:_NR, :]
        denom = jnp.maximum(jnp.sum(m), 1.0)
        px = jax.nn.sigmoid(ps[:, 0:1])
        py = jax.nn.sigmoid(ps[:, 1:2])
        pw = jnp.exp(ps[:, 2:3]) * aw
        ph = jnp.exp(ps[:, 3:4]) * ah
        b1x1 = px - pw * 0.5
        b1x2 = px + pw * 0.5
        b1y1 = py - ph * 0.5
        b1y2 = py + ph * 0.5
        b2x1 = tbx - tw * 0.5
        b2x2 = tbx + tw * 0.5
        b2y1 = tby - th * 0.5
        b2y2 = tby + th * 0.5
        iw = jnp.maximum(jnp.minimum(b1x2, b2x2) - jnp.maximum(b1x1, b2x1), 0.0)
        ih = jnp.maximum(jnp.minimum(b1y2, b2y2) - jnp.maximum(b1y1, b2y1), 0.0)
        inter = iw * ih
        w1 = b1x2 - b1x1
        h1 = b1y2 - b1y1
        w2 = b2x2 - b2x1
        h2 = b2y2 - b2y1
        union = w1 * h1 + w2 * h2 - inter + 1e-9
        iou = inter / union
        lbox = lbox + jnp.sum(m * (1.0 - iou)) / denom
        xc = ps[:, 5:_NO]
        cls_iota = lax.broadcasted_iota(
            jnp.int32, (_NR, _NUM_CLASS), 1).astype(jnp.float32)
        oh_cls = (cls_iota == c).astype(jnp.float32)
        e = jnp.maximum(xc, 0.0) - xc * oh_cls + jnp.log1p(jnp.exp(-jnp.abs(xc)))
        lcls = lcls + jnp.sum(m * e) / (denom * _NUM_CLASS)
        row_iota = lax.broadcasted_iota(jnp.int32, (_NR, 1), 0).astype(
            jnp.float32)
        key = jnp.where(m > 0.0, fidx, -(row_iota + 1.0))
        ones = jnp.ones((_NR, 1), jnp.float32)
        keyrow = lax.dot_general(ones, key, (((1,), (1,)), ((), ())),
                                 precision=lax.Precision.HIGHEST,
                                 preferred_element_type=jnp.float32)
        ii = lax.broadcasted_iota(jnp.int32, (_NR, _NR), 0)
        jj = lax.broadcasted_iota(jnp.int32, (_NR, _NR), 1)
        later_dup = jnp.where((keyrow == key) & (jj > ii), 1.0, 0.0)
        loser = jnp.max(later_dup, axis=1, keepdims=True)
        v = jnp.maximum(iou, 0.0)
        corr = jnp.sum(m * (1.0 - loser) * ps[:, 4:5] * v)
        lobj = lobj + (c_ref[0, 0] - corr) / ncell
    loss = 0.05 * lcls + lobj + 0.5 * lbox
    o_ref[:, :] = jnp.broadcast_to(loss, (1, 1))


def _final(target, c0, c1, c2, ps0, ps1, ps2):
    return pl.pallas_call(
        _final_body,
        out_shape=jax.ShapeDtypeStruct((1, 1), jnp.float32),
    )(target, c0, c1, c2, ps0, ps1, ps2)


def kernel(pred0, pred1, pred2, target):
    views = [p.reshape(_NSL[s], _GRIDS[s], _NO)
             for s, p in enumerate((pred0, pred1, pred2))]
    idxs = [i.reshape(_NRP) for i in _prep(target)]
    ps0, ps1, ps2 = _sc_gather(*views, idxs)
    s0, s1, s2 = _obj_sums(pred0, pred1, pred2)
    out = _final(target, s0, s1, s2, ps0, ps1, ps2)
    return out.reshape(1)


# speedup vs baseline: 1.0275x; 1.0017x over previous
import functools

import jax
import jax.numpy as jnp
from jax import lax
from jax.experimental import pallas as pl
from jax.experimental.pallas import tpu as pltpu
from jax.experimental.pallas import tpu_sc as plsc

_NUM_CLASS = 80
_STRIDES = (32.0, 16.0, 8.0)
_ANCHORS = (
    ((116.0, 90.0), (156.0, 198.0), (373.0, 326.0)),
    ((30.0, 61.0), (62.0, 45.0), (59.0, 119.0)),
    ((10.0, 13.0), (16.0, 30.0), (33.0, 23.0)),
)
_GRIDS = (19, 38, 76)
_B = 16
_NT = 128
_NO = 85
_NR = 3 * _NT
_NRP = 512
_NC, _NS = 2, 16
_NW = _NC * _NS
_RPW = _NRP // _NW


def _scale_targets(tgt, s):
    g = float(_GRIDS[s])
    stride = _STRIDES[s]
    b = tgt[:, 0:1]
    c = tgt[:, 1:2]
    gx = tgt[:, 2:3] * g
    gy = tgt[:, 3:4] * g
    tw = tgt[:, 4:5] * g
    th = tgt[:, 5:6] * g
    fx = jnp.floor(gx)
    fy = jnp.floor(gy)
    gi = jnp.clip(fx, 0.0, g - 1.0)
    gj = jnp.clip(fy, 0.0, g - 1.0)
    tbx = gx - fx
    tby = gy - fy
    ones = jnp.ones_like(b)
    per_anchor = []
    for a in range(3):
        aw = _ANCHORS[s][a][0] / stride
        ah = _ANCHORS[s][a][1] / stride
        rw = tw / aw
        rh = th / ah
        rmax = jnp.maximum(jnp.maximum(rw, 1.0 / rw), jnp.maximum(rh, 1.0 / rh))
        m = (rmax < 4.0).astype(jnp.float32)
        fidx = ((b * 3.0 + float(a)) * g + gj) * g + gi
        per_anchor.append((m, fidx, tbx, tby, tw, th, aw * ones, ah * ones, c,
                           b, a * ones, gj, gi))
    return [jnp.concatenate([pa[k] for pa in per_anchor], axis=0)
            for k in range(13)]


def _prep_body(tgt_ref, *orefs):
    tgt = tgt_ref[:, :]
    pad = jnp.zeros((_NRP - _NR, 1), jnp.float32)
    for s in range(3):
        g = float(_GRIDS[s])
        vals = _scale_targets(tgt, s)
        b, a, gj, gi = vals[9], vals[10], vals[11], vals[12]
        srow = (b * 3.0 + a) * g + gj
        for k, v in enumerate((srow, gi)):
            orefs[2 * s + k][:, :] = jnp.concatenate(
                [v, pad], axis=0).astype(jnp.int32)


def _prep(target):
    return pl.pallas_call(
        _prep_body,
        out_shape=[jax.ShapeDtypeStruct((_NRP, 1), jnp.int32)] * 6,
    )(target)


_NSL = (48 * 19, 48 * 38, 48 * 76)


def _sc_gather_body(t0, t1, t2, *rest):
    idx_refs = rest[:6]
    outs = rest[6:9]
    svs = rest[9:12]
    gvs = rest[12:15]
    rows = rest[15:18]
    semi, sem, semo = rest[18:21]
    wid = lax.axis_index("s") * _NC + lax.axis_index("c")
    tabs = (t0, t1, t2)
    base = wid * _RPW

    @pl.when(wid < _NR // _RPW)
    def _work():
        _sc_gather_work(tabs, idx_refs, outs, svs, gvs, rows, semi, sem, semo,
                        base)


def _sc_gather_work(tabs, idx_refs, outs, svs, gvs, rows, semi, sem, semo,
                    base):
    ijobs = []
    for s in range(3):
        ijobs.append(pltpu.async_copy(
            idx_refs[2 * s].at[pl.ds(base, _RPW)], svs[s], semi))
        ijobs.append(pltpu.async_copy(
            idx_refs[2 * s + 1].at[pl.ds(base, _RPW)], gvs[s], semi))
    for j in ijobs:
        j.wait()
    rjobs = []
    for s in range(3):
        ss = svs[s][...]
        gg = gvs[s][...]
        for r in range(_RPW):
            rjobs.append(pltpu.async_copy(
                tabs[s].at[ss[r], gg[r], :], rows[s].at[r, :], sem))
    for j in rjobs:
        j.wait()
    ojobs = []
    for s in range(3):
        ojobs.append(pltpu.async_copy(
            rows[s], outs[s].at[pl.ds(base, _RPW), :], semo))
    for j in ojobs:
        j.wait()


@functools.cache
def _sc_gather_kernel():
    return pl.kernel(
        _sc_gather_body,
        out_type=[jax.ShapeDtypeStruct((_NRP, _NO), jnp.float32)] * 3,
        mesh=plsc.VectorSubcoreMesh(core_axis_name="c", subcore_axis_name="s",
                                    num_cores=_NC, num_subcores=_NS),
        scratch_types=(
            [pltpu.VMEM((_RPW,), jnp.int32)] * 6
            + [pltpu.VMEM((_RPW, _NO), jnp.float32)] * 3
            + [pltpu.SemaphoreType.DMA] * 3
        ),
    )


def _sc_gather(p0, p1, p2, idxs):
    return _sc_gather_kernel()(p0, p1, p2, *idxs)


def _obj_body(p0_ref, p1a_ref, p1b_ref, p1c_ref, p2a_ref, p2b_ref,
              p2c_ref, s0_ref, s1_ref, s2_ref):
    step = pl.program_id(0)

    def _sp_sum(x4):
        return jnp.sum(jnp.maximum(x4, 0.0) + jnp.log1p(jnp.exp(-jnp.abs(x4))))

    x = p0_ref[0, :, :, :, :]
    part0 = _sp_sum(jnp.swapaxes(x, -1, -2)[:, :, 4, :])
    part1 = 0.0
    for r in (p1a_ref, p1b_ref, p1c_ref):
        x = r[0, 0, :, :, :]
        part1 = part1 + _sp_sum(jnp.swapaxes(x, -1, -2)[:, 4, :])
    part2 = 0.0
    for r in (p2a_ref, p2b_ref, p2c_ref):
        x = r[0, 0, :, :, :]
        part2 = part2 + _sp_sum(jnp.swapaxes(x, -1, -2)[:, 4, :])
    for s_ref, part in ((s0_ref, part0), (s1_ref, part1), (s2_ref, part2)):
        p11 = jnp.broadcast_to(part, (1, 1))

        @pl.when(step == 0)
        def _init(s_ref=s_ref, p11=p11):
            s_ref[:, :] = p11

        @pl.when(step != 0)
        def _acc(s_ref=s_ref, p11=p11):
            s_ref[:, :] += p11


def _obj_sums(pred0, pred1, pred2):
    specs = [pl.BlockSpec((1, 3, 19, 19, _NO), lambda t: (t, 0, 0, 0, 0))]
    for g in (38, 76):
        for a in range(3):
            specs.append(pl.BlockSpec(
                (1, 1, g, g, _NO),
                lambda t, a=a: (t, a, 0, 0, 0)))
    return pl.pallas_call(
        _obj_body,
        grid=(_B,),
        in_specs=specs,
        out_specs=[pl.BlockSpec((1, 1), lambda t: (0, 0))] * 3,
        out_shape=[jax.ShapeDtypeStruct((1, 1), jnp.float32)] * 3,
    )(pred0, pred1, pred1, pred1, pred2, pred2, pred2)


def _final_body(tgt_ref, c0_ref, c1_ref, c2_ref, ps0_ref, ps1_ref, ps2_ref,
                o_ref):
    tgt = tgt_ref[:, :]
    lcls = 0.0
    lobj = 0.0
    lbox = 0.0
    for s, c_ref, ps_ref in ((0, c0_ref, ps0_ref), (1, c1_ref, ps1_ref),
                             (2, c2_ref, ps2_ref)):
        ncell = float(_B * 3 * _GRIDS[s] * _GRIDS[s])
        m, fidx, tbx, tby, tw, th, aw, ah, c = _scale_targets(tgt, s)[:9]
        ps = ps_ref[0:_NR, :]
        denom = jnp.maximum(jnp.sum(m), 1.0)
        px = jax.nn.sigmoid(ps[:, 0:1])
        py = jax.nn.sigmoid(ps[:, 1:2])
        pw = jnp.exp(ps[:, 2:3]) * aw
        ph = jnp.exp(ps[:, 3:4]) * ah
        b1x1 = px - pw * 0.5
        b1x2 = px + pw * 0.5
        b1y1 = py - ph * 0.5
        b1y2 = py + ph * 0.5
        b2x1 = tbx - tw * 0.5
        b2x2 = tbx + tw * 0.5
        b2y1 = tby - th * 0.5
        b2y2 = tby + th * 0.5
        iw = jnp.maximum(jnp.minimum(b1x2, b2x2) - jnp.maximum(b1x1, b2x1), 0.0)
        ih = jnp.maximum(jnp.minimum(b1y2, b2y2) - jnp.maximum(b1y1, b2y1), 0.0)
        inter = iw * ih
        w1 = b1x2 - b1x1
        h1 = b1y2 - b1y1
        w2 = b2x2 - b2x1
        h2 = b2y2 - b2y1
        union = w1 * h1 + w2 * h2 - inter + 1e-9
        iou = inter / union
        lbox = lbox + jnp.sum(m * (1.0 - iou)) / denom
        xc = ps[:, 5:_NO]
        cls_iota = lax.broadcasted_iota(
            jnp.int32, (_NR, _NUM_CLASS), 1).astype(jnp.float32)
        oh_cls = (cls_iota == c).astype(jnp.float32)
        e = jnp.maximum(xc, 0.0) - xc * oh_cls + jnp.log1p(jnp.exp(-jnp.abs(xc)))
        lcls = lcls + jnp.sum(m * e) / (denom * _NUM_CLASS)
        row_iota = lax.broadcasted_iota(jnp.int32, (_NR, 1), 0).astype(
            jnp.float32)
        key = jnp.where(m > 0.0, fidx, -(row_iota + 1.0))
        ones = jnp.ones((_NR, 1), jnp.float32)
        keyrow = lax.dot_general(ones, key, (((1,), (1,)), ((), ())),
                                 precision=lax.Precision.HIGHEST,
                                 preferred_element_type=jnp.float32)
        ii = lax.broadcasted_iota(jnp.int32, (_NR, _NR), 0)
        jj = lax.broadcasted_iota(jnp.int32, (_NR, _NR), 1)
        later_dup = jnp.where((keyrow == key) & (jj > ii), 1.0, 0.0)
        loser = jnp.max(later_dup, axis=1, keepdims=True)
        v = jnp.maximum(iou, 0.0)
        corr = jnp.sum(m * (1.0 - loser) * ps[:, 4:5] * v)
        lobj = lobj + (c_ref[0, 0] - corr) / ncell
    loss = 0.05 * lcls + lobj + 0.5 * lbox
    o_ref[:, :] = jnp.broadcast_to(loss, (1, 1))


def _final(target, c0, c1, c2, ps0, ps1, ps2):
    return pl.pallas_call(
        _final_body,
        out_shape=jax.ShapeDtypeStruct((1, 1), jnp.float32),
    )(target, c0, c1, c2, ps0, ps1, ps2)


def kernel(pred0, pred1, pred2, target):
    views = [p.reshape(_NSL[s], _GRIDS[s], _NO)
             for s, p in enumerate((pred0, pred1, pred2))]
    idxs = [i.reshape(_NRP) for i in _prep(target)]
    ps0, ps1, ps2 = _sc_gather(*views, idxs)
    s0, s1, s2 = _obj_sums(pred0, pred1, pred2)
    out = _final(target, s0, s1, s2, ps0, ps1, ps2)
    return out.reshape(1)
